# Initial kernel scaffold; baseline (speedup 1.0000x reference)
import jax
import jax.numpy as jnp
from jax.experimental import pallas as pl


def kernel(x, edge_index, W1, b1, W2, b2):
    raise NotImplementedError("write your pallas kernel here")


# speedup vs baseline: 211.3291x; 211.3291x over previous
import functools

import jax
import jax.numpy as jnp
from jax import lax
from jax.experimental import pallas as pl
from jax.experimental.pallas import tpu as pltpu
from jax.experimental.pallas import tpu_sc as plsc

NC = 2
NS = 16
NW = NC * NS
L = 16


_SC_PARAMS = pltpu.CompilerParams(needs_layout_passes=False)


def _wid():
    return lax.axis_index("s") * NC + lax.axis_index("c")


def _make_deg_kernel(E, Np):
    ept = E // NW
    nv = ept // L
    mesh = plsc.VectorSubcoreMesh(core_axis_name="c", subcore_axis_name="s")

    @functools.partial(
        pl.kernel,
        out_type=jax.ShapeDtypeStruct((NW, Np), jnp.float32),
        mesh=mesh,
        scratch_types=[
            pltpu.VMEM((ept,), jnp.int32),
            pltpu.VMEM((Np,), jnp.float32),
        ],
        compiler_params=_SC_PARAMS,
    )
    def deg_k(dst_hbm, zeros_hbm, out_hbm, dst_v, acc_v):
        wid = _wid()
        pltpu.sync_copy(dst_hbm.at[pl.ds(wid * ept, ept)], dst_v)
        pltpu.sync_copy(zeros_hbm, acc_v)
        ones = jnp.ones((L,), jnp.float32)

        @plsc.parallel_loop(0, ept, L, unroll=8)
        def _(off):
            d = dst_v[pl.ds(off, L)]
            plsc.addupdate_scatter(acc_v, [d], ones)

        pltpu.sync_copy(acc_v, out_hbm.at[wid])

    return deg_k


def _make_gs1_kernel(E, Np):
    ept = E // NW
    nv = ept // L
    mesh = plsc.VectorSubcoreMesh(core_axis_name="c", subcore_axis_name="s")

    @functools.partial(
        pl.kernel,
        out_type=jax.ShapeDtypeStruct((NW, Np), jnp.float32),
        mesh=mesh,
        scratch_types=[
            pltpu.VMEM((ept,), jnp.int32),
            pltpu.VMEM((ept,), jnp.int32),
            pltpu.VMEM((Np,), jnp.float32),
            pltpu.VMEM((Np,), jnp.float32),
        ],
        compiler_params=_SC_PARAMS,
    )
    def gs1_k(src_hbm, dst_hbm, val_hbm, zeros_hbm, out_hbm,
              src_v, dst_v, val_v, acc_v):
        wid = _wid()
        pltpu.sync_copy(val_hbm, val_v)
        pltpu.sync_copy(src_hbm.at[pl.ds(wid * ept, ept)], src_v)
        pltpu.sync_copy(dst_hbm.at[pl.ds(wid * ept, ept)], dst_v)
        pltpu.sync_copy(zeros_hbm, acc_v)

        @plsc.parallel_loop(0, ept, L, unroll=8)
        def _(off):
            s = src_v[pl.ds(off, L)]
            d = dst_v[pl.ds(off, L)]
            vals = plsc.load_gather(val_v, [s])
            plsc.addupdate_scatter(acc_v, [d], vals)

        pltpu.sync_copy(acc_v, out_hbm.at[wid])

    return gs1_k


def _make_gs2_kernel(E, Np):
    ept = E // NW
    nv = ept // L
    mesh = plsc.VectorSubcoreMesh(core_axis_name="c", subcore_axis_name="s")

    @functools.partial(
        pl.kernel,
        out_type=(
            jax.ShapeDtypeStruct((NW, Np), jnp.float32),
            jax.ShapeDtypeStruct((NW, Np), jnp.float32),
        ),
        mesh=mesh,
        scratch_types=[
            pltpu.VMEM((ept,), jnp.int32),
            pltpu.VMEM((ept,), jnp.int32),
            pltpu.VMEM((Np,), jnp.float32),
            pltpu.VMEM((Np,), jnp.float32),
            pltpu.VMEM((Np,), jnp.float32),
            pltpu.VMEM((Np,), jnp.float32),
        ],
        compiler_params=_SC_PARAMS,
    )
    def gs2_k(src_hbm, dst_hbm, va_hbm, vb_hbm, zeros_hbm, outa_hbm, outb_hbm,
              src_v, dst_v, va_v, vb_v, acca_v, accb_v):
        wid = _wid()
        pltpu.sync_copy(va_hbm, va_v)
        pltpu.sync_copy(vb_hbm, vb_v)
        pltpu.sync_copy(src_hbm.at[pl.ds(wid * ept, ept)], src_v)
        pltpu.sync_copy(dst_hbm.at[pl.ds(wid * ept, ept)], dst_v)
        pltpu.sync_copy(zeros_hbm, acca_v)
        pltpu.sync_copy(zeros_hbm, accb_v)

        @plsc.parallel_loop(0, ept, L, unroll=8)
        def _(off):
            s = src_v[pl.ds(off, L)]
            d = dst_v[pl.ds(off, L)]
            a = plsc.load_gather(va_v, [s])
            plsc.addupdate_scatter(acca_v, [d], a)
            b = plsc.load_gather(vb_v, [s])
            plsc.addupdate_scatter(accb_v, [d], b)
        pltpu.sync_copy(acca_v, outa_hbm.at[wid])
        pltpu.sync_copy(accb_v, outb_hbm.at[wid])

    return gs2_k


def _tc_prep(pd_ref, x_ref, dinv_ref, v1_ref):
    deg = jnp.sum(pd_ref[...], axis=0) + 1.0
    dinv = lax.rsqrt(deg)
    dinv_ref[...] = dinv
    v1_ref[...] = dinv * x_ref[...]


def _tc_mid(ps_ref, dinv_ref, v1_ref, w_ref, v2a_ref, v2b_ref):
    s1 = jnp.sum(ps_ref[...], axis=0)
    dinv = dinv_ref[...]
    t1 = dinv * (s1 + v1_ref[...])
    h0 = jnp.zeros_like(t1)
    h1 = jnp.zeros_like(t1)
    for k in range(16):
        z = jnp.maximum(t1 * w_ref[0, k] + w_ref[1, k], 0.0)
        h0 = h0 + z * w_ref[2, k]
        h1 = h1 + z * w_ref[3, k]
    v2a_ref[...] = dinv * h0
    v2b_ref[...] = dinv * h1


def _tc_out(psa_ref, psb_ref, dinv_ref, v2a_ref, v2b_ref, b2_ref, outa_ref, outb_ref):
    dinv = dinv_ref[...]
    o0 = dinv * (jnp.sum(psa_ref[...], axis=0) + v2a_ref[...]) + b2_ref[0]
    o1 = dinv * (jnp.sum(psb_ref[...], axis=0) + v2b_ref[...]) + b2_ref[1]
    m = jnp.maximum(o0, o1)
    lse = m + jnp.log(jnp.exp(o0 - m) + jnp.exp(o1 - m))
    outa_ref[...] = o0 - lse
    outb_ref[...] = o1 - lse


def _vspec():
    return pl.BlockSpec(memory_space=pltpu.VMEM)


def _sspec():
    return pl.BlockSpec(memory_space=pltpu.SMEM)


def kernel(x, edge_index, W1, b1, W2, b2):
    N = x.shape[0]
    E = edge_index.shape[1]
    Np = ((N + 1023) // 1024) * 1024
    R = Np // 128

    src = edge_index[0]
    dst = edge_index[1]
    xp = jnp.pad(x[:, 0], (0, Np - N)).reshape(R, 128)

    deg_k = _make_deg_kernel(E, Np)
    gs1_k = _make_gs1_kernel(E, Np)
    gs2_k = _make_gs2_kernel(E, Np)

    zeros_np = jnp.zeros((Np,), jnp.float32)
    deg_parts = deg_k(dst, zeros_np)

    dinv, v1 = pl.pallas_call(
        _tc_prep,
        out_shape=(
            jax.ShapeDtypeStruct((R, 128), jnp.float32),
            jax.ShapeDtypeStruct((R, 128), jnp.float32),
        ),
        in_specs=[_vspec(), _vspec()],
        out_specs=(_vspec(), _vspec()),
    )(deg_parts.reshape(NW, R, 128), xp)

    s1_parts = gs1_k(src, dst, v1.reshape(Np), zeros_np)

    wpack = jnp.stack([W1[0], b1, W2[:, 0], W2[:, 1]])
    v2a, v2b = pl.pallas_call(
        _tc_mid,
        out_shape=(
            jax.ShapeDtypeStruct((R, 128), jnp.float32),
            jax.ShapeDtypeStruct((R, 128), jnp.float32),
        ),
        in_specs=[_vspec(), _vspec(), _vspec(), _sspec()],
        out_specs=(_vspec(), _vspec()),
    )(s1_parts.reshape(NW, R, 128), dinv, v1, wpack)

    s2a_parts, s2b_parts = gs2_k(src, dst, v2a.reshape(Np), v2b.reshape(Np), zeros_np)

    outa, outb = pl.pallas_call(
        _tc_out,
        out_shape=(
            jax.ShapeDtypeStruct((R, 128), jnp.float32),
            jax.ShapeDtypeStruct((R, 128), jnp.float32),
        ),
        in_specs=[_vspec(), _vspec(), _vspec(), _vspec(), _vspec(), _sspec()],
        out_specs=(_vspec(), _vspec()),
    )(s2a_parts.reshape(NW, R, 128), s2b_parts.reshape(NW, R, 128),
      dinv, v2a, v2b, b2)

    out = jnp.stack([outa.reshape(Np), outb.reshape(Np)], axis=1)
    return out[:N]
